# baseline (speedup 1.0000x reference)
import functools

import jax
import jax.numpy as jnp
from jax import lax
from jax.experimental import pallas as pl
from jax.experimental.pallas import tpu as pltpu
from jax.experimental.pallas import tpu_sc as plsc

B = 16
N = 32768
D = 512
MAX_NODES = 4096

R = 4096
GRID = N // R

NTILES = 32
CHUNK = (B * MAX_NODES) // NTILES
CP = CHUNK + 16
FILL = -1000000000.0


def _tc_body(bid_ref, x_hbm, ga_ref, w_ref, bias_ref, temp_ref,
             sim_ref, offs_ref, pn_ref, pnn_ref, cacc_ref, xbuf, sem):
    i = pl.program_id(0)
    slot = lax.rem(i, 2)

    @pl.when(i == 0)
    def _init():
        pltpu.make_async_copy(x_hbm.at[pl.ds(0, R), :],
                              xbuf.at[0], sem.at[0]).start()
        pn = lax.dot_general(ga_ref[...], w_ref[...],
                             (((1,), (1,)), ((), ())),
                             preferred_element_type=jnp.float32)
        pn = pn + bias_ref[...]
        pn_ref[...] = pn.astype(jnp.bfloat16)
        pnn_ref[...] = jnp.sum(pn * pn, axis=1, keepdims=True)
        cacc_ref[...] = jnp.zeros_like(cacc_ref)

    @pl.when(i + 1 < GRID)
    def _prefetch():
        nslot = lax.rem(i + 1, 2)
        pltpu.make_async_copy(x_hbm.at[pl.ds((i + 1) * R, R), :],
                              xbuf.at[nslot], sem.at[nslot]).start()

    ids = bid_ref[0, :, :]
    jota = lax.broadcasted_iota(jnp.int32, (B, R), 0)
    onehot = (ids == jota).astype(jnp.float32)
    cacc_ref[...] += (ids < jota).astype(jnp.float32)

    pltpu.make_async_copy(x_hbm.at[pl.ds(i * R, R), :],
                          xbuf.at[slot], sem.at[slot]).wait()
    x = xbuf[slot].astype(jnp.bfloat16)
    sq = x * x
    ones8 = jnp.ones((8, D), jnp.bfloat16)
    rn2 = lax.dot_general(ones8, sq, (((1,), (1,)), ((), ())),
                          preferred_element_type=jnp.float32)
    xpT = lax.dot_general(pn_ref[...], x, (((1,), (1,)), ((), ())),
                          preferred_element_type=jnp.float32)
    t = onehot * (pnn_ref[...] - 2.0 * xpT)
    d2 = rn2[0:1, :] + jnp.sum(t, axis=0, keepdims=True)
    d2 = jnp.maximum(d2, 0.0)
    sim_ref[0, :, :] = -jnp.sqrt(d2) / temp_ref[0, 0]

    @pl.when(i == GRID - 1)
    def _fin():
        offs_col = jnp.sum(cacc_ref[...], axis=1, keepdims=True)
        offs_row = offs_col.reshape(1, B)
        full = jnp.concatenate(
            [offs_row, jnp.full((1, 128 - B), float(N), jnp.float32)], axis=1)
        offs_ref[...] = jnp.broadcast_to(full, (8, 128)).astype(jnp.int32)


@jax.jit
def _tc_call(bid3, x, graph_attr, W, bias2, temp2):
    return pl.pallas_call(
        _tc_body,
        grid=(GRID,),
        in_specs=[
            pl.BlockSpec((1, 1, R), lambda i: (i, 0, 0)),
            pl.BlockSpec(memory_space=pl.ANY),
            pl.BlockSpec((B, D), lambda i: (0, 0)),
            pl.BlockSpec((D, D), lambda i: (0, 0)),
            pl.BlockSpec((1, D), lambda i: (0, 0)),
            pl.BlockSpec(memory_space=pltpu.SMEM),
        ],
        out_specs=[
            pl.BlockSpec((1, 1, R), lambda i: (i, 0, 0)),
            pl.BlockSpec((8, 128), lambda i: (0, 0)),
        ],
        out_shape=[
            jax.ShapeDtypeStruct((GRID, 1, R), jnp.float32),
            jax.ShapeDtypeStruct((8, 128), jnp.int32),
        ],
        scratch_shapes=[
            pltpu.VMEM((B, D), jnp.bfloat16),
            pltpu.VMEM((B, 1), jnp.float32),
            pltpu.VMEM((B, R), jnp.float32),
            pltpu.VMEM((2, R, D), jnp.float32),
            pltpu.SemaphoreType.DMA((2,)),
        ],
        compiler_params=pltpu.CompilerParams(
            dimension_semantics=("arbitrary",)),
    )(bid3, x, graph_attr, W, bias2, temp2)


def _sc_body(sim_hbm, offs_hbm, out_hbm, offs_v, buf, obuf):
    c = lax.axis_index("c")
    s = lax.axis_index("s")
    wid = s * 2 + c
    pltpu.sync_copy(offs_hbm.at[0], offs_v)
    gb = wid // 2
    j0 = (wid % 2) * CHUNK
    bvec = jnp.full((16,), gb, jnp.int32)
    start = jnp.max(plsc.load_gather(offs_v, [bvec])) + j0
    end = jnp.max(plsc.load_gather(offs_v, [bvec + 1]))
    astart = jnp.minimum((start // 8) * 8, N - CP)
    pltpu.sync_copy(sim_hbm.at[pl.ds(astart, CP)], buf)
    sh = start - astart
    lanes = lax.iota(jnp.int32, 16)
    for k in range(CHUNK // 16):
        idx = sh + k * 16 + lanes
        idxc = jnp.minimum(idx, CP - 1)
        v = plsc.load_gather(buf, [idxc])
        valid = (astart + idx) < end
        obuf[pl.ds(k * 16, 16)] = jnp.where(valid, v, FILL)
    pltpu.sync_copy(obuf, out_hbm.at[pl.ds(wid * CHUNK, CHUNK)])


@jax.jit
def _sc_call(sim, offs):
    fn = functools.partial(
        pl.kernel,
        out_type=jax.ShapeDtypeStruct((B * MAX_NODES,), jnp.float32),
        mesh=plsc.VectorSubcoreMesh(core_axis_name="c", subcore_axis_name="s"),
        compiler_params=pltpu.CompilerParams(needs_layout_passes=False),
        scratch_types=[
            pltpu.VMEM((128,), jnp.int32),
            pltpu.VMEM((CP,), jnp.float32),
            pltpu.VMEM((CHUNK,), jnp.float32),
        ],
    )(_sc_body)
    return fn(sim, offs)


def kernel(x, graph_attr, batch_ids, W, b, temp):
    bid3 = batch_ids.astype(jnp.int32).reshape(GRID, 1, R)
    sim3, offs = _tc_call(bid3, x, graph_attr, W,
                          b.reshape(1, D),
                          temp.reshape(1, 1))
    dense = _sc_call(sim3.reshape(N), offs)
    return dense.reshape(B, MAX_NODES, 1)


# speedup vs baseline: 1.0002x; 1.0002x over previous
import functools

import jax
import jax.numpy as jnp
from jax import lax
from jax.experimental import pallas as pl
from jax.experimental.pallas import tpu as pltpu
from jax.experimental.pallas import tpu_sc as plsc

B = 16
N = 32768
D = 512
MAX_NODES = 4096

R = 4096
GRID = N // R

NTILES = 32
CHUNK = (B * MAX_NODES) // NTILES
CP = CHUNK + 16
FILL = -1000000000.0


def _tc_body(bid_ref, x_hbm, ga_ref, w_ref, bias_ref, temp_ref,
             sim_ref, offs_ref, pn_ref, pnn_ref, cacc_ref, xbuf, sem):
    i = pl.program_id(0)
    slot = lax.rem(i, 2)

    H = R // 2

    @pl.when(i == 0)
    def _init():
        pltpu.make_async_copy(x_hbm.at[pl.ds(0, H), :],
                              xbuf.at[0, 0:H], sem.at[0, 0]).start()
        pltpu.make_async_copy(x_hbm.at[pl.ds(H, H), :],
                              xbuf.at[0, H:R], sem.at[0, 1]).start()
        pn = lax.dot_general(ga_ref[...], w_ref[...],
                             (((1,), (1,)), ((), ())),
                             preferred_element_type=jnp.float32)
        pn = pn + bias_ref[...]
        pn_ref[...] = pn.astype(jnp.bfloat16)
        pnn_ref[...] = jnp.sum(pn * pn, axis=1, keepdims=True)
        cacc_ref[...] = jnp.zeros_like(cacc_ref)

    @pl.when(i + 1 < GRID)
    def _prefetch():
        nslot = lax.rem(i + 1, 2)
        pltpu.make_async_copy(x_hbm.at[pl.ds((i + 1) * R, H), :],
                              xbuf.at[nslot, 0:H], sem.at[nslot, 0]).start()
        pltpu.make_async_copy(x_hbm.at[pl.ds((i + 1) * R + H, H), :],
                              xbuf.at[nslot, H:R], sem.at[nslot, 1]).start()

    ids = bid_ref[0, :, :]
    jota = lax.broadcasted_iota(jnp.int32, (B, R), 0)
    onehot = (ids == jota).astype(jnp.float32)
    cacc_ref[...] += (ids < jota).astype(jnp.float32)

    pltpu.make_async_copy(x_hbm.at[pl.ds(i * R, H), :],
                          xbuf.at[slot, 0:H], sem.at[slot, 0]).wait()
    pltpu.make_async_copy(x_hbm.at[pl.ds(i * R + H, H), :],
                          xbuf.at[slot, H:R], sem.at[slot, 1]).wait()
    x = xbuf[slot].astype(jnp.bfloat16)
    sq = x * x
    ones8 = jnp.ones((8, D), jnp.bfloat16)
    rn2 = lax.dot_general(ones8, sq, (((1,), (1,)), ((), ())),
                          preferred_element_type=jnp.float32)
    xpT = lax.dot_general(pn_ref[...], x, (((1,), (1,)), ((), ())),
                          preferred_element_type=jnp.float32)
    t = onehot * (pnn_ref[...] - 2.0 * xpT)
    d2 = rn2[0:1, :] + jnp.sum(t, axis=0, keepdims=True)
    d2 = jnp.maximum(d2, 0.0)
    sim_ref[0, :, :] = -jnp.sqrt(d2) / temp_ref[0, 0]

    @pl.when(i == GRID - 1)
    def _fin():
        offs_col = jnp.sum(cacc_ref[...], axis=1, keepdims=True)
        offs_row = offs_col.reshape(1, B)
        full = jnp.concatenate(
            [offs_row, jnp.full((1, 128 - B), float(N), jnp.float32)], axis=1)
        offs_ref[...] = jnp.broadcast_to(full, (8, 128)).astype(jnp.int32)


@jax.jit
def _tc_call(bid3, x, graph_attr, W, bias2, temp2):
    return pl.pallas_call(
        _tc_body,
        grid=(GRID,),
        in_specs=[
            pl.BlockSpec((1, 1, R), lambda i: (i, 0, 0)),
            pl.BlockSpec(memory_space=pl.ANY),
            pl.BlockSpec((B, D), lambda i: (0, 0)),
            pl.BlockSpec((D, D), lambda i: (0, 0)),
            pl.BlockSpec((1, D), lambda i: (0, 0)),
            pl.BlockSpec(memory_space=pltpu.SMEM),
        ],
        out_specs=[
            pl.BlockSpec((1, 1, R), lambda i: (i, 0, 0)),
            pl.BlockSpec((8, 128), lambda i: (0, 0)),
        ],
        out_shape=[
            jax.ShapeDtypeStruct((GRID, 1, R), jnp.float32),
            jax.ShapeDtypeStruct((8, 128), jnp.int32),
        ],
        scratch_shapes=[
            pltpu.VMEM((B, D), jnp.bfloat16),
            pltpu.VMEM((B, 1), jnp.float32),
            pltpu.VMEM((B, R), jnp.float32),
            pltpu.VMEM((2, R, D), jnp.float32),
            pltpu.SemaphoreType.DMA((2, 2)),
        ],
        compiler_params=pltpu.CompilerParams(
            dimension_semantics=("arbitrary",)),
    )(bid3, x, graph_attr, W, bias2, temp2)


def _sc_body(sim_hbm, offs_hbm, out_hbm, offs_v, buf, obuf):
    c = lax.axis_index("c")
    s = lax.axis_index("s")
    wid = s * 2 + c
    pltpu.sync_copy(offs_hbm.at[0], offs_v)
    gb = wid // 2
    j0 = (wid % 2) * CHUNK
    bvec = jnp.full((16,), gb, jnp.int32)
    start = jnp.max(plsc.load_gather(offs_v, [bvec])) + j0
    end = jnp.max(plsc.load_gather(offs_v, [bvec + 1]))
    astart = jnp.minimum((start // 8) * 8, N - CP)
    pltpu.sync_copy(sim_hbm.at[pl.ds(astart, CP)], buf)
    sh = start - astart
    lanes = lax.iota(jnp.int32, 16)
    for k in range(CHUNK // 16):
        idx = sh + k * 16 + lanes
        idxc = jnp.minimum(idx, CP - 1)
        v = plsc.load_gather(buf, [idxc])
        valid = (astart + idx) < end
        obuf[pl.ds(k * 16, 16)] = jnp.where(valid, v, FILL)
    pltpu.sync_copy(obuf, out_hbm.at[pl.ds(wid * CHUNK, CHUNK)])


@jax.jit
def _sc_call(sim, offs):
    fn = functools.partial(
        pl.kernel,
        out_type=jax.ShapeDtypeStruct((B * MAX_NODES,), jnp.float32),
        mesh=plsc.VectorSubcoreMesh(core_axis_name="c", subcore_axis_name="s"),
        compiler_params=pltpu.CompilerParams(needs_layout_passes=False),
        scratch_types=[
            pltpu.VMEM((128,), jnp.int32),
            pltpu.VMEM((CP,), jnp.float32),
            pltpu.VMEM((CHUNK,), jnp.float32),
        ],
    )(_sc_body)
    return fn(sim, offs)


def kernel(x, graph_attr, batch_ids, W, b, temp):
    bid3 = batch_ids.astype(jnp.int32).reshape(GRID, 1, R)
    sim3, offs = _tc_call(bid3, x, graph_attr, W,
                          b.reshape(1, D),
                          temp.reshape(1, 1))
    dense = _sc_call(sim3.reshape(N), offs)
    return dense.reshape(B, MAX_NODES, 1)


# speedup vs baseline: 1.0191x; 1.0189x over previous
import functools

import jax
import jax.numpy as jnp
from jax import lax
from jax.experimental import pallas as pl
from jax.experimental.pallas import tpu as pltpu
from jax.experimental.pallas import tpu_sc as plsc

B = 16
N = 32768
D = 512
MAX_NODES = 4096

R = 4096
GRID = N // R

NTILES = 32
CHUNK = (B * MAX_NODES) // NTILES
CP = CHUNK + 16
FILL = -1000000000.0


def _tc_body(bid_ref, x_ref, ga_ref, w_ref, bias_ref, temp_ref,
             sim_ref, offs_ref, pn_ref, pnn_ref, cacc_ref):
    i = pl.program_id(0)

    @pl.when(i == 0)
    def _init():
        pn = lax.dot_general(ga_ref[...], w_ref[...],
                             (((1,), (1,)), ((), ())),
                             preferred_element_type=jnp.float32)
        pn = pn + bias_ref[...]
        pn_ref[...] = pn
        pnn_ref[...] = jnp.sum(pn * pn, axis=1, keepdims=True)
        cacc_ref[...] = jnp.zeros_like(cacc_ref)

    ids = bid_ref[0, :, :]
    jota = lax.broadcasted_iota(jnp.int32, (B, R), 0)
    onehot = (ids == jota).astype(jnp.float32)
    cacc_ref[...] += (ids < jota).astype(jnp.float32)

    x = x_ref[...]
    sq = x * x
    ones8 = jnp.ones((8, D), jnp.float32)
    rn2 = lax.dot_general(ones8, sq, (((1,), (1,)), ((), ())),
                          preferred_element_type=jnp.float32)
    xpT = lax.dot_general(pn_ref[...], x, (((1,), (1,)), ((), ())),
                          preferred_element_type=jnp.float32)
    t = onehot * (pnn_ref[...] - 2.0 * xpT)
    d2 = rn2[0:1, :] + jnp.sum(t, axis=0, keepdims=True)
    d2 = jnp.maximum(d2, 0.0)
    sim_ref[0, :, :] = -jnp.sqrt(d2) / temp_ref[0, 0]

    @pl.when(i == GRID - 1)
    def _fin():
        offs_col = jnp.sum(cacc_ref[...], axis=1, keepdims=True)
        offs_row = offs_col.reshape(1, B)
        full = jnp.concatenate(
            [offs_row, jnp.full((1, 128 - B), float(N), jnp.float32)], axis=1)
        offs_ref[...] = jnp.broadcast_to(full, (8, 128)).astype(jnp.int32)


@jax.jit
def _tc_call(bid3, x, graph_attr, W, bias2, temp2):
    return pl.pallas_call(
        _tc_body,
        grid=(GRID,),
        in_specs=[
            pl.BlockSpec((1, 1, R), lambda i: (i, 0, 0)),
            pl.BlockSpec((R, D), lambda i: (i, 0)),
            pl.BlockSpec((B, D), lambda i: (0, 0)),
            pl.BlockSpec((D, D), lambda i: (0, 0)),
            pl.BlockSpec((1, D), lambda i: (0, 0)),
            pl.BlockSpec(memory_space=pltpu.SMEM),
        ],
        out_specs=[
            pl.BlockSpec((1, 1, R), lambda i: (i, 0, 0)),
            pl.BlockSpec((8, 128), lambda i: (0, 0)),
        ],
        out_shape=[
            jax.ShapeDtypeStruct((GRID, 1, R), jnp.float32),
            jax.ShapeDtypeStruct((8, 128), jnp.int32),
        ],
        scratch_shapes=[
            pltpu.VMEM((B, D), jnp.float32),
            pltpu.VMEM((B, 1), jnp.float32),
            pltpu.VMEM((B, R), jnp.float32),
        ],
    )(bid3, x, graph_attr, W, bias2, temp2)


def _sc_body(sim_hbm, offs_hbm, out_hbm, offs_v, buf, obuf):
    c = lax.axis_index("c")
    s = lax.axis_index("s")
    wid = s * 2 + c
    pltpu.sync_copy(offs_hbm.at[0], offs_v)
    gb = wid // 2
    j0 = (wid % 2) * CHUNK
    bvec = jnp.full((16,), gb, jnp.int32)
    start = jnp.max(plsc.load_gather(offs_v, [bvec])) + j0
    end = jnp.max(plsc.load_gather(offs_v, [bvec + 1]))
    astart = jnp.minimum((start // 8) * 8, N - CP)
    pltpu.sync_copy(sim_hbm.at[pl.ds(astart, CP)], buf)
    sh = start - astart
    lanes = lax.iota(jnp.int32, 16)
    for k in range(CHUNK // 16):
        idx = sh + k * 16 + lanes
        idxc = jnp.minimum(idx, CP - 1)
        v = plsc.load_gather(buf, [idxc])
        valid = (astart + idx) < end
        obuf[pl.ds(k * 16, 16)] = jnp.where(valid, v, FILL)
    pltpu.sync_copy(obuf, out_hbm.at[pl.ds(wid * CHUNK, CHUNK)])


@jax.jit
def _sc_call(sim, offs):
    fn = functools.partial(
        pl.kernel,
        out_type=jax.ShapeDtypeStruct((B * MAX_NODES,), jnp.float32),
        mesh=plsc.VectorSubcoreMesh(core_axis_name="c", subcore_axis_name="s"),
        compiler_params=pltpu.CompilerParams(needs_layout_passes=False),
        scratch_types=[
            pltpu.VMEM((128,), jnp.int32),
            pltpu.VMEM((CP,), jnp.float32),
            pltpu.VMEM((CHUNK,), jnp.float32),
        ],
    )(_sc_body)
    return fn(sim, offs)


def kernel(x, graph_attr, batch_ids, W, b, temp):
    bid3 = batch_ids.astype(jnp.int32).reshape(GRID, 1, R)
    sim3, offs = _tc_call(bid3, x, graph_attr, W,
                          b.reshape(1, D),
                          temp.reshape(1, 1))
    dense = _sc_call(sim3.reshape(N), offs)
    return dense.reshape(B, MAX_NODES, 1)
